# baseline (speedup 1.0000x reference)
import functools

import numpy as np
import jax
import jax.numpy as jnp
from jax import lax
from jax.experimental import pallas as pl
from jax.experimental.pallas import tpu as pltpu
from jax.experimental.pallas import tpu_sc as plsc

RES = 32
CH = 4
PS = 3
PD = 108
B = 4
S = 2048
DIN = 64
NPTS = B * S
NTAP = PS ** 3
PAYW = 8
ROWS = NPTS * NTAP
GRID_ROWS = B * RES ** 3
HALF = GRID_ROWS // 2

NTILES = 32
PT = ROWS // NTILES
CHUNK = 128
NCHUNK = PT // CHUNK
IDX_MAJ = ROWS // CHUNK

MLP_BLK = 1024
PROJ_BLK = 4096

_PMAP = np.zeros((PD, NTAP * PAYW), np.float32)
for _j in range(PD):
    _PMAP[_j, PAYW * (_j // 4) + (_j // 27)] = 1.0
_MVEC = np.zeros((NTAP * PAYW,), np.float32)
for _j in range(NTAP):
    _MVEC[PAYW * (_j // 4) + 4] += 1.0


def _gelu(v):
    return 0.5 * v * (1.0 + lax.erf(v * np.float32(0.7071067811865476)))


def _mlp_body(pos_ref, x_ref, w1_ref, b1_ref, w2_ref, b2_ref, wf_ref, bf_ref,
              u_ref, idx_ref):
    h = jnp.dot(x_ref[...].astype(jnp.bfloat16), w1_ref[...],
                preferred_element_type=jnp.float32)
    h = _gelu(h + b1_ref[...])
    h = jnp.dot(h.astype(jnp.bfloat16), w2_ref[...],
                preferred_element_type=jnp.float32)
    h = _gelu(h + b2_ref[...])
    u = jnp.dot(h.astype(jnp.bfloat16), wf_ref[...],
                preferred_element_type=jnp.float32)
    u_ref[...] = u + bf_ref[...]

    blk = pos_ref.shape[0]
    pid = pl.program_id(0) * blk + lax.broadcasted_iota(jnp.int32, (blk, 1), 0)
    vx = (pos_ref[:, 0:1] * RES).astype(jnp.int32)
    vy = (pos_ref[:, 1:2] * RES).astype(jnp.int32)
    vz = (pos_ref[:, 2:3] * RES).astype(jnp.int32)
    o = lax.broadcasted_iota(jnp.int32, (blk, NTAP), 1)
    cx = jnp.clip(vx + (o // 3) % 3 - 1, 0, RES - 1)
    cy = jnp.clip(vy + o // 9 - 1, 0, RES - 1)
    cz = jnp.clip(vz + o % 3 - 1, 0, RES - 1)
    half = ((pid >> 11) & 1) << 15
    idx_ref[...] = (cx * RES + cy) * RES + cz + half


def _run_mlp(positions2, x2, W1, b1, W2, b2, Wf2, bf2, interpret=False):
    nblk = NPTS // MLP_BLK
    full = lambda i: (0, 0)
    return pl.pallas_call(
        _mlp_body,
        grid=(nblk,),
        in_specs=[
            pl.BlockSpec((MLP_BLK, 3), lambda i: (i, 0)),
            pl.BlockSpec((MLP_BLK, DIN), lambda i: (i, 0)),
            pl.BlockSpec((DIN, PD * 8), full),
            pl.BlockSpec((1, PD * 8), full),
            pl.BlockSpec((PD * 8, PD * 4), full),
            pl.BlockSpec((1, PD * 4), full),
            pl.BlockSpec((PD * 4, NTAP * PAYW), full),
            pl.BlockSpec((1, NTAP * PAYW), full),
        ],
        out_specs=[
            pl.BlockSpec((MLP_BLK, NTAP * PAYW), lambda i: (i, 0)),
            pl.BlockSpec((MLP_BLK, NTAP), lambda i: (i, 0)),
        ],
        out_shape=[
            jax.ShapeDtypeStruct((NPTS, NTAP * PAYW), jnp.float32),
            jax.ShapeDtypeStruct((NPTS, NTAP), jnp.int32),
        ],
        interpret=interpret,
    )(positions2, x2, W1, b1, W2, b2, Wf2, bf2)


def _sc_scatter_body(u_hbm, idx_hbm, zeros_hbm, out_hbm, pay_v, idx_v, grid_sh):
    c = lax.axis_index("c")
    s = lax.axis_index("s")
    tile = c * 16 + s
    zrows = HALF // 16
    pltpu.sync_copy(zeros_hbm.at[pl.ds(s * zrows, zrows)],
                    grid_sh.at[pl.ds(s * zrows, zrows)])
    row0 = tile * PT
    pltpu.sync_copy(u_hbm.at[pl.ds(row0, PT)], pay_v)
    pltpu.sync_copy(idx_hbm.at[tile], idx_v)
    plsc.subcore_barrier()

    def group(g, carry):
        for k in range(6):
            j = g * 6 + k
            pltpu.sync_copy(pay_v.at[pl.ds(j * CHUNK, CHUNK)],
                            grid_sh.at[idx_v.at[j]], add=True)
        return carry

    lax.fori_loop(0, NCHUNK // 6, group, 0)
    plsc.subcore_barrier()
    out0 = c * HALF + s * zrows
    pltpu.sync_copy(grid_sh.at[pl.ds(s * zrows, zrows)],
                    out_hbm.at[pl.ds(out0, zrows)])


def _make_sc_scatter():
    return pl.kernel(
        _sc_scatter_body,
        mesh=plsc.VectorSubcoreMesh(core_axis_name="c", subcore_axis_name="s"),
        out_type=jax.ShapeDtypeStruct((GRID_ROWS, PAYW), jnp.float32),
        compiler_params=pltpu.CompilerParams(use_tc_tiling_on_sc=False),
        scratch_types=[
            pltpu.VMEM((PT, PAYW), jnp.float32),
            pltpu.VMEM((NCHUNK, CHUNK), jnp.int32),
            pltpu.VMEM_SHARED((HALF, PAYW), jnp.float32),
        ],
    )


def _proj_body(g_ref, wt_ref, bp_ref, o_ref):
    wblk = PROJ_BLK // 16
    x = g_ref[...]
    a = jnp.broadcast_to(x[:, None, :], (wblk, 16, 128)).reshape(PROJ_BLK, 128)
    r = lax.broadcasted_iota(jnp.int32, (PROJ_BLK, 128), 0)
    j = lax.broadcasted_iota(jnp.int32, (PROJ_BLK, 128), 1)
    m = (j // PAYW == r % 16).astype(jnp.float32)
    o_ref[...] = (jnp.dot(a * m, wt_ref[...],
                          preferred_element_type=jnp.float32) + bp_ref[...])


def _run_proj(grid_wide, Wtile, bp2, interpret=False):
    nblk = GRID_ROWS // PROJ_BLK
    wblk = PROJ_BLK // 16
    return pl.pallas_call(
        _proj_body,
        grid=(nblk,),
        in_specs=[
            pl.BlockSpec((wblk, 128), lambda i: (i, 0)),
            pl.BlockSpec((128, PD), lambda i: (0, 0)),
            pl.BlockSpec((1, PD), lambda i: (0, 0)),
        ],
        out_specs=pl.BlockSpec((PROJ_BLK, PD), lambda i: (i, 0)),
        out_shape=jax.ShapeDtypeStruct((GRID_ROWS, PD), jnp.float32),
        interpret=interpret,
    )(grid_wide, Wtile, bp2)


def kernel(positions, x, W1, b1, W2, b2, Wf, bf, Wp, bp):
    pmap = jnp.asarray(_PMAP)
    Wf2 = Wf @ pmap
    bf2 = (bf @ pmap + jnp.asarray(_MVEC)).reshape(1, -1)
    positions2 = positions.reshape(NPTS, 3)
    x2 = x.reshape(NPTS, DIN)

    u, idx = _run_mlp(positions2, x2, W1.astype(jnp.bfloat16),
                      b1.reshape(1, -1), W2.astype(jnp.bfloat16),
                      b2.reshape(1, -1), Wf2.astype(jnp.bfloat16), bf2)
    u_rows = u.reshape(ROWS, PAYW)
    idx_rows = idx.reshape(NTILES, NCHUNK, CHUNK)
    zeros = jnp.zeros((HALF, PAYW), jnp.float32)

    grid8 = _make_sc_scatter()(u_rows, idx_rows, zeros)

    Wp8 = jnp.concatenate([Wp, jnp.zeros((PAYW - CH - 1, PD), Wp.dtype)], axis=0)
    Wtile = jnp.tile(Wp8, (16, 1))
    out = _run_proj(grid8.reshape(GRID_ROWS * PAYW // 128, 128), Wtile,
                    bp.reshape(1, -1))
    return out.reshape(B, RES, RES, RES, PD)


# speedup vs baseline: 1.0097x; 1.0097x over previous
import functools

import numpy as np
import jax
import jax.numpy as jnp
from jax import lax
from jax.experimental import pallas as pl
from jax.experimental.pallas import tpu as pltpu
from jax.experimental.pallas import tpu_sc as plsc

RES = 32
CH = 4
PS = 3
PD = 108
B = 4
S = 2048
DIN = 64
NPTS = B * S
NTAP = PS ** 3
PAYW = 8
ROWS = NPTS * NTAP
GRID_ROWS = B * RES ** 3
HALF = GRID_ROWS // 2

NTILES = 32
PT = ROWS // NTILES
CHUNK = 128
NCHUNK = PT // CHUNK
IDX_MAJ = ROWS // CHUNK

MLP_BLK = 1024
PROJ_BLK = 4096

_PMAP = np.zeros((PD, NTAP * PAYW), np.float32)
for _j in range(PD):
    _PMAP[_j, PAYW * (_j // 4) + (_j // 27)] = 1.0
_MVEC = np.zeros((NTAP * PAYW,), np.float32)
for _j in range(NTAP):
    _MVEC[PAYW * (_j // 4) + 4] += 1.0


def _gelu(v):
    return 0.5 * v * (1.0 + lax.erf(v * np.float32(0.7071067811865476)))


def _mlp_body(pos_ref, x_ref, w1_ref, b1_ref, w2_ref, b2_ref, wf_ref, bf_ref,
              u_ref, idx_ref):
    h = jnp.dot(x_ref[...], w1_ref[...], preferred_element_type=jnp.float32)
    h = _gelu(h + b1_ref[...])
    h = jnp.dot(h, w2_ref[...], preferred_element_type=jnp.float32)
    h = _gelu(h + b2_ref[...])
    u = jnp.dot(h, wf_ref[...], preferred_element_type=jnp.float32)
    u_ref[...] = u + bf_ref[...]

    blk = pos_ref.shape[0]
    pid = pl.program_id(0) * blk + lax.broadcasted_iota(jnp.int32, (blk, 1), 0)
    vx = (pos_ref[:, 0:1] * RES).astype(jnp.int32)
    vy = (pos_ref[:, 1:2] * RES).astype(jnp.int32)
    vz = (pos_ref[:, 2:3] * RES).astype(jnp.int32)
    o = lax.broadcasted_iota(jnp.int32, (blk, NTAP), 1)
    cx = jnp.clip(vx + (o // 3) % 3 - 1, 0, RES - 1)
    cy = jnp.clip(vy + o // 9 - 1, 0, RES - 1)
    cz = jnp.clip(vz + o % 3 - 1, 0, RES - 1)
    half = ((pid >> 11) & 1) << 15
    idx_ref[...] = (cx * RES + cy) * RES + cz + half


def _run_mlp(positions2, x2, W1, b1, W2, b2, Wf2, bf2, interpret=False):
    nblk = NPTS // MLP_BLK
    full = lambda i: (0, 0)
    return pl.pallas_call(
        _mlp_body,
        grid=(nblk,),
        in_specs=[
            pl.BlockSpec((MLP_BLK, 3), lambda i: (i, 0)),
            pl.BlockSpec((MLP_BLK, DIN), lambda i: (i, 0)),
            pl.BlockSpec((DIN, PD * 8), full),
            pl.BlockSpec((1, PD * 8), full),
            pl.BlockSpec((PD * 8, PD * 4), full),
            pl.BlockSpec((1, PD * 4), full),
            pl.BlockSpec((PD * 4, NTAP * PAYW), full),
            pl.BlockSpec((1, NTAP * PAYW), full),
        ],
        out_specs=[
            pl.BlockSpec((MLP_BLK, NTAP * PAYW), lambda i: (i, 0)),
            pl.BlockSpec((MLP_BLK, NTAP), lambda i: (i, 0)),
        ],
        out_shape=[
            jax.ShapeDtypeStruct((NPTS, NTAP * PAYW), jnp.float32),
            jax.ShapeDtypeStruct((NPTS, NTAP), jnp.int32),
        ],
        interpret=interpret,
    )(positions2, x2, W1, b1, W2, b2, Wf2, bf2)


def _sc_scatter_body(u_hbm, idx_hbm, zeros_hbm, out_hbm, pay_v, idx_v, grid_sh):
    c = lax.axis_index("c")
    s = lax.axis_index("s")
    tile = c * 16 + s
    zrows = HALF // 16
    pltpu.sync_copy(zeros_hbm.at[pl.ds(s * zrows, zrows)],
                    grid_sh.at[pl.ds(s * zrows, zrows)])
    row0 = tile * PT
    pltpu.sync_copy(u_hbm.at[pl.ds(row0, PT)], pay_v)
    pltpu.sync_copy(idx_hbm.at[tile], idx_v)
    plsc.subcore_barrier()

    def group(g, carry):
        for k in range(6):
            j = g * 6 + k
            pltpu.sync_copy(pay_v.at[pl.ds(j * CHUNK, CHUNK)],
                            grid_sh.at[idx_v.at[j]], add=True)
        return carry

    lax.fori_loop(0, NCHUNK // 6, group, 0)
    plsc.subcore_barrier()
    out0 = c * HALF + s * zrows
    pltpu.sync_copy(grid_sh.at[pl.ds(s * zrows, zrows)],
                    out_hbm.at[pl.ds(out0, zrows)])


def _make_sc_scatter():
    return pl.kernel(
        _sc_scatter_body,
        mesh=plsc.VectorSubcoreMesh(core_axis_name="c", subcore_axis_name="s"),
        out_type=jax.ShapeDtypeStruct((GRID_ROWS, PAYW), jnp.float32),
        compiler_params=pltpu.CompilerParams(use_tc_tiling_on_sc=False),
        scratch_types=[
            pltpu.VMEM((PT, PAYW), jnp.float32),
            pltpu.VMEM((NCHUNK, CHUNK), jnp.int32),
            pltpu.VMEM_SHARED((HALF, PAYW), jnp.float32),
        ],
    )


def _proj_body(g_ref, wt_ref, bp_ref, o_ref):
    wblk = PROJ_BLK // 16
    x = g_ref[...]
    a = jnp.broadcast_to(x[:, None, :], (wblk, 16, 128)).reshape(PROJ_BLK, 128)
    r = lax.broadcasted_iota(jnp.int32, (PROJ_BLK, 128), 0)
    j = lax.broadcasted_iota(jnp.int32, (PROJ_BLK, 128), 1)
    m = (j // PAYW == r % 16).astype(jnp.float32)
    o_ref[...] = (jnp.dot(a * m, wt_ref[...],
                          preferred_element_type=jnp.float32) + bp_ref[...])


def _run_proj(grid_wide, Wtile, bp2, interpret=False):
    nblk = GRID_ROWS // PROJ_BLK
    wblk = PROJ_BLK // 16
    return pl.pallas_call(
        _proj_body,
        grid=(nblk,),
        in_specs=[
            pl.BlockSpec((wblk, 128), lambda i: (i, 0)),
            pl.BlockSpec((128, PD), lambda i: (0, 0)),
            pl.BlockSpec((1, PD), lambda i: (0, 0)),
        ],
        out_specs=pl.BlockSpec((PROJ_BLK, PD), lambda i: (i, 0)),
        out_shape=jax.ShapeDtypeStruct((GRID_ROWS, PD), jnp.float32),
        interpret=interpret,
    )(grid_wide, Wtile, bp2)


def kernel(positions, x, W1, b1, W2, b2, Wf, bf, Wp, bp):
    pmap = jnp.asarray(_PMAP)
    Wf2 = Wf @ pmap
    bf2 = (bf @ pmap + jnp.asarray(_MVEC)).reshape(1, -1)
    positions2 = positions.reshape(NPTS, 3)
    x2 = x.reshape(NPTS, DIN)

    u, idx = _run_mlp(positions2, x2, W1, b1.reshape(1, -1), W2,
                      b2.reshape(1, -1), Wf2, bf2)
    u_rows = u.reshape(ROWS, PAYW)
    idx_rows = idx.reshape(NTILES, NCHUNK, CHUNK)
    zeros = jnp.zeros((HALF, PAYW), jnp.float32)

    grid8 = _make_sc_scatter()(u_rows, idx_rows, zeros)

    Wp8 = jnp.concatenate([Wp, jnp.zeros((PAYW - CH - 1, PD), Wp.dtype)], axis=0)
    Wtile = jnp.tile(Wp8, (16, 1))
    out = _run_proj(grid8.reshape(GRID_ROWS * PAYW // 128, 128), Wtile,
                    bp.reshape(1, -1))
    return out.reshape(B, RES, RES, RES, PD)


# speedup vs baseline: 1.0522x; 1.0420x over previous
import functools

import numpy as np
import jax
import jax.numpy as jnp
from jax import lax
from jax.experimental import pallas as pl
from jax.experimental.pallas import tpu as pltpu
from jax.experimental.pallas import tpu_sc as plsc

RES = 32
CH = 4
PS = 3
PD = 108
B = 4
S = 2048
DIN = 64
NPTS = B * S
NTAP = PS ** 3
PAYW = 8
ROWS = NPTS * NTAP
GRID_ROWS = B * RES ** 3
HALF = GRID_ROWS // 2

NTILES = 32
PT = ROWS // NTILES
CHUNK = 128
NCHUNK = PT // CHUNK
IDX_MAJ = ROWS // CHUNK

MLP_BLK = 1024
PROJ_BLK = 4096

_PMAP = np.zeros((PD, NTAP * PAYW), np.float32)
for _j in range(PD):
    _PMAP[_j, PAYW * (_j // 4) + (_j // 27)] = 1.0
_MVEC = np.zeros((NTAP * PAYW,), np.float32)
for _j in range(NTAP):
    _MVEC[PAYW * (_j // 4) + 4] += 1.0


def _gelu(v):
    return 0.5 * v * (1.0 + lax.erf(v * np.float32(0.7071067811865476)))


def _mlp_body(pos_ref, x_ref, w1_ref, b1_ref, w2_ref, b2_ref, wf_ref, bf_ref,
              u_ref, idx_ref):
    h = jnp.dot(x_ref[...], w1_ref[...], preferred_element_type=jnp.float32)
    h = _gelu(h + b1_ref[...])
    h = jnp.dot(h, w2_ref[...], preferred_element_type=jnp.float32)
    h = _gelu(h + b2_ref[...])
    u = jnp.dot(h, wf_ref[...], preferred_element_type=jnp.float32)
    u_ref[...] = u + bf_ref[...]

    blk = pos_ref.shape[0]
    pid = pl.program_id(0) * blk + lax.broadcasted_iota(jnp.int32, (blk, 1), 0)
    vx = (pos_ref[:, 0:1] * RES).astype(jnp.int32)
    vy = (pos_ref[:, 1:2] * RES).astype(jnp.int32)
    vz = (pos_ref[:, 2:3] * RES).astype(jnp.int32)
    o = lax.broadcasted_iota(jnp.int32, (blk, NTAP), 1)
    cx = jnp.clip(vx + (o // 3) % 3 - 1, 0, RES - 1)
    cy = jnp.clip(vy + o // 9 - 1, 0, RES - 1)
    cz = jnp.clip(vz + o % 3 - 1, 0, RES - 1)
    half = ((pid >> 11) & 1) << 15
    idx_ref[...] = (cx * RES + cy) * RES + cz + half


def _run_mlp(positions2, x2, W1, b1, W2, b2, Wf2, bf2, interpret=False):
    nblk = NPTS // MLP_BLK
    full = lambda i: (0, 0)
    return pl.pallas_call(
        _mlp_body,
        grid=(nblk,),
        in_specs=[
            pl.BlockSpec((MLP_BLK, 3), lambda i: (i, 0)),
            pl.BlockSpec((MLP_BLK, DIN), lambda i: (i, 0)),
            pl.BlockSpec((DIN, PD * 8), full),
            pl.BlockSpec((1, PD * 8), full),
            pl.BlockSpec((PD * 8, PD * 4), full),
            pl.BlockSpec((1, PD * 4), full),
            pl.BlockSpec((PD * 4, NTAP * PAYW), full),
            pl.BlockSpec((1, NTAP * PAYW), full),
        ],
        out_specs=[
            pl.BlockSpec((MLP_BLK, NTAP * PAYW), lambda i: (i, 0)),
            pl.BlockSpec((MLP_BLK, NTAP), lambda i: (i, 0)),
        ],
        out_shape=[
            jax.ShapeDtypeStruct((NPTS, NTAP * PAYW), jnp.float32),
            jax.ShapeDtypeStruct((NPTS, NTAP), jnp.int32),
        ],
        interpret=interpret,
    )(positions2, x2, W1, b1, W2, b2, Wf2, bf2)


def _sc_scatter_body(u_hbm, idx_hbm, zeros_hbm, out_hbm, pay_v, idx_v, grid_sh,
                     sem_in, sem_sc):
    c = lax.axis_index("c")
    s = lax.axis_index("s")
    tile = c * 16 + s
    zrows = HALF // 16
    row0 = tile * PT
    d0 = pltpu.async_copy(u_hbm.at[pl.ds(row0, PT)], pay_v, sem_in)
    d1 = pltpu.async_copy(idx_hbm.at[tile], idx_v, sem_in)
    d2 = pltpu.async_copy(zeros_hbm.at[pl.ds(s * zrows, zrows)],
                          grid_sh.at[pl.ds(s * zrows, zrows)], sem_in)
    d0.wait()
    d1.wait()
    d2.wait()
    plsc.subcore_barrier()

    def group(g, carry):
        ds = []
        for k in range(6):
            j = g * 6 + k
            ds.append(pltpu.async_copy(pay_v.at[pl.ds(j * CHUNK, CHUNK)],
                                       grid_sh.at[idx_v.at[j]], sem_sc,
                                       add=True))
        for d in ds:
            d.wait()
        return carry

    lax.fori_loop(0, NCHUNK // 6, group, 0)
    plsc.subcore_barrier()
    out0 = c * HALF + s * zrows
    pltpu.sync_copy(grid_sh.at[pl.ds(s * zrows, zrows)],
                    out_hbm.at[pl.ds(out0, zrows)])


def _make_sc_scatter():
    return pl.kernel(
        _sc_scatter_body,
        mesh=plsc.VectorSubcoreMesh(core_axis_name="c", subcore_axis_name="s"),
        out_type=jax.ShapeDtypeStruct((GRID_ROWS, PAYW), jnp.float32),
        compiler_params=pltpu.CompilerParams(use_tc_tiling_on_sc=False),
        scratch_types=[
            pltpu.VMEM((PT, PAYW), jnp.float32),
            pltpu.VMEM((NCHUNK, CHUNK), jnp.int32),
            pltpu.VMEM_SHARED((HALF, PAYW), jnp.float32),
            pltpu.SemaphoreType.DMA,
            pltpu.SemaphoreType.DMA,
        ],
    )


def _proj_body(g_ref, wt_ref, bp_ref, o_ref):
    wblk = PROJ_BLK // 16
    x = g_ref[...]
    a = jnp.broadcast_to(x[:, None, :], (wblk, 16, 128)).reshape(PROJ_BLK, 128)
    r = lax.broadcasted_iota(jnp.int32, (PROJ_BLK, 128), 0)
    j = lax.broadcasted_iota(jnp.int32, (PROJ_BLK, 128), 1)
    m = (j // PAYW == r % 16).astype(jnp.float32)
    o_ref[...] = (jnp.dot(a * m, wt_ref[...],
                          preferred_element_type=jnp.float32) + bp_ref[...])


def _run_proj(grid_wide, Wtile, bp2, interpret=False):
    nblk = GRID_ROWS // PROJ_BLK
    wblk = PROJ_BLK // 16
    return pl.pallas_call(
        _proj_body,
        grid=(nblk,),
        in_specs=[
            pl.BlockSpec((wblk, 128), lambda i: (i, 0)),
            pl.BlockSpec((128, PD), lambda i: (0, 0)),
            pl.BlockSpec((1, PD), lambda i: (0, 0)),
        ],
        out_specs=pl.BlockSpec((PROJ_BLK, PD), lambda i: (i, 0)),
        out_shape=jax.ShapeDtypeStruct((GRID_ROWS, PD), jnp.float32),
        interpret=interpret,
    )(grid_wide, Wtile, bp2)


def kernel(positions, x, W1, b1, W2, b2, Wf, bf, Wp, bp):
    pmap = jnp.asarray(_PMAP)
    Wf2 = Wf @ pmap
    bf2 = (bf @ pmap + jnp.asarray(_MVEC)).reshape(1, -1)
    positions2 = positions.reshape(NPTS, 3)
    x2 = x.reshape(NPTS, DIN)

    u, idx = _run_mlp(positions2, x2, W1, b1.reshape(1, -1), W2,
                      b2.reshape(1, -1), Wf2, bf2)
    u_rows = u.reshape(ROWS, PAYW)
    idx_rows = idx.reshape(NTILES, NCHUNK, CHUNK)
    zeros = jnp.zeros((HALF, PAYW), jnp.float32)

    grid8 = _make_sc_scatter()(u_rows, idx_rows, zeros)

    Wp8 = jnp.concatenate([Wp, jnp.zeros((PAYW - CH - 1, PD), Wp.dtype)], axis=0)
    Wtile = jnp.tile(Wp8, (16, 1))
    out = _run_proj(grid8.reshape(GRID_ROWS * PAYW // 128, 128), Wtile,
                    bp.reshape(1, -1))
    return out.reshape(B, RES, RES, RES, PD)
